# baseline (speedup 1.0000x reference)
import jax
import jax.numpy as jnp
from jax import lax
from jax.experimental import pallas as pl
from jax.experimental.pallas import tpu as pltpu
from jax.experimental.pallas import tpu_sc as plsc

_N, _E, _D = 10000, 320000, 128
_NG, _NCLS = 64, 10
_NP = 10240
_NC, _NS = 2, 16
_NW = _NC * _NS
_CH = 128
_NCHUNK = 80
_EPW = _NCHUNK * _CH
_RPT = _NP // _NS
_RB = 1024
_NBLK = _NP // _RB

_mesh = plsc.VectorSubcoreMesh(core_axis_name="c", subcore_axis_name="s")


_NIDX = _NCHUNK + 3


def _make_sc_body(with_deg):
    def body(*args):
        if with_deg:
            (table, edges, zrows, zdeg, nsum_out, deg_out,
             sb0, sb1, sb2, sb3, db0, db1, db2, db3, rows0, ones_v,
             acc_sh, deg_sh, isem, jsem, gsem) = args
        else:
            (table, edges, zrows, nsum_out,
             sb0, sb1, sb2, sb3, db0, db1, db2, db3, rows0,
             acc_sh, isem, jsem, gsem) = args
        srcb = [sb0, sb1, sb2, sb3]
        dstb = [db0, db1, db2, db3]
        c = lax.axis_index("c")
        s = lax.axis_index("s")
        wid = c * _NS + s

        def i_issue(ch, slot):
            pltpu.async_copy(edges.at[wid, ch, 0], srcb[slot],
                             jsem.at[slot])
            pltpu.async_copy(edges.at[wid, ch, 1], dstb[slot],
                             isem.at[slot])

        def i_wait(slot):
            pltpu.make_async_copy(edges.at[wid, 0, 0], srcb[slot],
                                  jsem.at[slot]).wait()
            pltpu.make_async_copy(edges.at[wid, 0, 1], dstb[slot],
                                  isem.at[slot]).wait()

        def g_issue(islot, rslot):
            pltpu.async_copy(table.at[dstb[islot]], rows.at[rslot],
                             gsem.at[rslot])

        def g_wait(rslot):
            pltpu.make_async_copy(table.at[db0], rows.at[rslot],
                                  gsem.at[rslot]).wait()

        pltpu.sync_copy(zrows.at[pl.ds(s * _RPT, _RPT)],
                        acc_sh.at[pl.ds(s * _RPT, _RPT)])
        if with_deg:
            pltpu.sync_copy(zdeg.at[pl.ds(s * _RPT, _RPT)],
                            deg_sh.at[pl.ds(s * _RPT, _RPT)])
            for k in range(_CH // 16):
                ones_v[pl.ds(k * 16, 16)] = jnp.full((16,), 1.0, jnp.float32)
        plsc.subcore_barrier()

        def step(j, carry):
            pltpu.sync_copy(edges.at[wid, j, 0], srcb[0])
            pltpu.sync_copy(edges.at[wid, j, 1], dstb[0])
            pltpu.async_copy(table.at[dstb[0]], rows0, gsem.at[0]).wait()
            pltpu.sync_copy(rows0, acc_sh.at[srcb[0]], add=True)
            if with_deg:
                pltpu.sync_copy(ones_v, deg_sh.at[srcb[0]], add=True)
            return carry

        lax.fori_loop(0, _NCHUNK, step, 0)

        plsc.subcore_barrier()
        pltpu.sync_copy(acc_sh.at[pl.ds(s * _RPT, _RPT)],
                        nsum_out.at[c, pl.ds(s * _RPT, _RPT)])
        if with_deg:
            pltpu.sync_copy(deg_sh.at[pl.ds(s * _RPT, _RPT)],
                            deg_out.at[c, pl.ds(s * _RPT, _RPT)])

    return body


_IDX_SCRATCH = [pltpu.VMEM((_CH,), jnp.int32)] * 8

_seg_deg = pl.kernel(
    _make_sc_body(True),
    out_type=[jax.ShapeDtypeStruct((_NC, _NP, _D), jnp.float32),
              jax.ShapeDtypeStruct((_NC, _NP), jnp.float32)],
    mesh=_mesh,
    scratch_types=_IDX_SCRATCH +
                  [pltpu.VMEM((_CH, _D), jnp.float32),
                   pltpu.VMEM((_CH,), jnp.float32),
                   pltpu.VMEM_SHARED((_NP, _D), jnp.float32),
                   pltpu.VMEM_SHARED((_NP,), jnp.float32),
                   pltpu.SemaphoreType.DMA((4,)),
                   pltpu.SemaphoreType.DMA((4,)),
                   pltpu.SemaphoreType.DMA((2,))],
)

_seg = pl.kernel(
    _make_sc_body(False),
    out_type=[jax.ShapeDtypeStruct((_NC, _NP, _D), jnp.float32)],
    mesh=_mesh,
    scratch_types=_IDX_SCRATCH +
                  [pltpu.VMEM((_CH, _D), jnp.float32),
                   pltpu.VMEM_SHARED((_NP, _D), jnp.float32),
                   pltpu.SemaphoreType.DMA((4,)),
                   pltpu.SemaphoreType.DMA((4,)),
                   pltpu.SemaphoreType.DMA((2,))],
)


_DN_NT = (((1,), (1,)), ((), ()))
_DN_NN = (((1,), (0,)), ((), ()))


def _mm3_body(x_ref, wg_ref, wl_ref, ws_ref, hg_ref, hl_ref, hs_ref):
    xb = x_ref[...]
    hg_ref[...] = lax.dot_general(xb, wg_ref[...], _DN_NT,
                                  preferred_element_type=jnp.float32)
    hl_ref[...] = lax.dot_general(xb, wl_ref[...], _DN_NT,
                                  preferred_element_type=jnp.float32)
    hs_ref[...] = lax.dot_general(xb, ws_ref[...], _DN_NT,
                                  preferred_element_type=jnp.float32)


_mm3 = pl.pallas_call(
    _mm3_body,
    grid=(_NBLK,),
    in_specs=[pl.BlockSpec((_RB, _D), lambda i: (i, 0)),
              pl.BlockSpec((_D, _D), lambda i: (0, 0)),
              pl.BlockSpec((_D, _D), lambda i: (0, 0)),
              pl.BlockSpec((_D, _D), lambda i: (0, 0))],
    out_specs=[pl.BlockSpec((_RB, _D), lambda i: (i, 0))] * 3,
    out_shape=[jax.ShapeDtypeStruct((_NP, _D), jnp.float32)] * 3,
)


def _layer_epilogue(hg_ref, hs_ref, nsump_ref, degp_ref, b_ref):
    ns = nsump_ref[...]
    nsum = ns[0] + ns[1]
    dp = degp_ref[...]
    deg = dp[0] + dp[1]
    inv = 1.0 / jnp.maximum(deg, 1.0)
    mask = (deg > 0.0).astype(jnp.float32)
    pre = hg_ref[...] + b_ref[...] + mask * (nsum * inv + hs_ref[...])
    return jnp.where(pre > 0.0, pre, jnp.exp(jnp.minimum(pre, 0.0)) - 1.0)


def _post_mm3_body(hg_ref, hs_ref, nsump_ref, degp_ref, b_ref,
                   wg_ref, wl_ref, ws_ref, hg2_ref, hl2_ref, hs2_ref):
    h1 = _layer_epilogue(hg_ref, hs_ref, nsump_ref, degp_ref, b_ref)
    hg2_ref[...] = lax.dot_general(h1, wg_ref[...], _DN_NT,
                                   preferred_element_type=jnp.float32)
    hl2_ref[...] = lax.dot_general(h1, wl_ref[...], _DN_NT,
                                   preferred_element_type=jnp.float32)
    hs2_ref[...] = lax.dot_general(h1, ws_ref[...], _DN_NT,
                                   preferred_element_type=jnp.float32)


_post_mm3 = pl.pallas_call(
    _post_mm3_body,
    grid=(_NBLK,),
    in_specs=[pl.BlockSpec((_RB, _D), lambda i: (i, 0)),
              pl.BlockSpec((_RB, _D), lambda i: (i, 0)),
              pl.BlockSpec((_NC, _RB, _D), lambda i: (0, i, 0)),
              pl.BlockSpec((_NC, _RB, 1), lambda i: (0, i, 0)),
              pl.BlockSpec((1, _D), lambda i: (0, 0)),
              pl.BlockSpec((_D, _D), lambda i: (0, 0)),
              pl.BlockSpec((_D, _D), lambda i: (0, 0)),
              pl.BlockSpec((_D, _D), lambda i: (0, 0))],
    out_specs=[pl.BlockSpec((_RB, _D), lambda i: (i, 0))] * 3,
    out_shape=[jax.ShapeDtypeStruct((_NP, _D), jnp.float32)] * 3,
)


def _final_body(hg_ref, hs_ref, nsump_ref, degp_ref, b_ref, batch_ref,
                wc_ref, bc_ref, out_ref, sums_s, cnts_s):
    i = pl.program_id(0)

    @pl.when(i == 0)
    def _():
        sums_s[...] = jnp.zeros_like(sums_s)
        cnts_s[...] = jnp.zeros_like(cnts_s)

    h2 = _layer_epilogue(hg_ref, hs_ref, nsump_ref, degp_ref, b_ref)
    bvec = batch_ref[...][0]
    oh = (lax.broadcasted_iota(jnp.int32, (_NG, _RB), 0)
          == jnp.broadcast_to(bvec, (_NG, _RB))).astype(jnp.float32)
    sums_s[...] += lax.dot_general(oh, h2, _DN_NN,
                                   preferred_element_type=jnp.float32)
    cnts_s[...] += jnp.broadcast_to(
        jnp.sum(oh, axis=1, keepdims=True), (_NG, _D))

    @pl.when(i == _NBLK - 1)
    def _():
        g = sums_s[...] / jnp.maximum(cnts_s[...], 1.0)
        out_ref[...] = lax.dot_general(g, wc_ref[...], _DN_NT,
                                       preferred_element_type=jnp.float32) \
            + bc_ref[...]


_final = pl.pallas_call(
    _final_body,
    grid=(_NBLK,),
    in_specs=[pl.BlockSpec((_RB, _D), lambda i: (i, 0)),
              pl.BlockSpec((_RB, _D), lambda i: (i, 0)),
              pl.BlockSpec((_NC, _RB, _D), lambda i: (0, i, 0)),
              pl.BlockSpec((_NC, _RB, 1), lambda i: (0, i, 0)),
              pl.BlockSpec((1, _D), lambda i: (0, 0)),
              pl.BlockSpec((1, 1, _RB), lambda i: (i, 0, 0)),
              pl.BlockSpec((_NCLS, _D), lambda i: (0, 0)),
              pl.BlockSpec((1, _NCLS), lambda i: (0, 0))],
    out_specs=pl.BlockSpec((_NG, _NCLS), lambda i: (0, 0)),
    out_shape=jax.ShapeDtypeStruct((_NG, _NCLS), jnp.float32),
    scratch_shapes=[pltpu.VMEM((_NG, _D), jnp.float32),
                    pltpu.VMEM((_NG, _D), jnp.float32)],
)


def kernel(x, edge_index, batch, Wg1, Wl1, Ws1, b1, Wg2, Wl2, Ws2, b2, Wc, bc):
    src = edge_index[0].astype(jnp.int32)
    dst = edge_index[1].astype(jnp.int32)
    pad = _NW * _EPW - _E
    srcp = jnp.concatenate(
        [src, jnp.full((pad,), _N, jnp.int32)]).reshape(_NW, _NCHUNK, _CH)
    dstp = jnp.concatenate(
        [dst, jnp.zeros((pad,), jnp.int32)]).reshape(_NW, _NCHUNK, _CH)
    srcp = jnp.concatenate(
        [srcp, jnp.full((_NW, _NIDX - _NCHUNK, _CH), _N, jnp.int32)], axis=1)
    dstp = jnp.concatenate(
        [dstp, jnp.zeros((_NW, _NIDX - _NCHUNK, _CH), jnp.int32)], axis=1)
    edges = jnp.stack([srcp, dstp], axis=2)
    xp = jnp.pad(x, ((0, _NP - _N), (0, 0)))
    batchp = jnp.concatenate(
        [batch.astype(jnp.int32),
         jnp.full((_NP - _N,), _NG, jnp.int32)]).reshape(_NBLK, 1, _RB)
    zrows = jnp.zeros((_NP, _D), jnp.float32)
    zdeg = jnp.zeros((_NP,), jnp.float32)
    b1r = b1.reshape(1, _D)
    b2r = b2.reshape(1, _D)
    bcr = bc.reshape(1, _NCLS)

    hg1, hl1, hs1 = _mm3(xp, Wg1, Wl1, Ws1)
    nsum1, degp = _seg_deg(hl1, edges, zrows, zdeg)
    degp3 = degp.reshape(_NC, _NP, 1)
    hg2, hl2, hs2 = _post_mm3(hg1, hs1, nsum1, degp3, b1r, Wg2, Wl2, Ws2)
    (nsum2,) = _seg(hl2, edges, zrows)
    return _final(hg2, hs2, nsum2, degp3, b2r, batchp, Wc, bcr)


# speedup vs baseline: 1.6050x; 1.6050x over previous
import jax
import jax.numpy as jnp
from jax import lax
from jax.experimental import pallas as pl
from jax.experimental.pallas import tpu as pltpu
from jax.experimental.pallas import tpu_sc as plsc

_N, _E, _D = 10000, 320000, 128
_NG, _NCLS = 64, 10
_NP = 10240
_NC, _NS = 2, 16
_NW = _NC * _NS
_CH = 128
_NCHUNK = 79
_EPW = _NCHUNK * _CH
_RPT = _NP // _NS
_RB = 1024
_NBLK = _NP // _RB

_mesh = plsc.VectorSubcoreMesh(core_axis_name="c", subcore_axis_name="s")


_NIDX = _NCHUNK + 3


def _sc_seg_deg_body(table, srcs, dsts, zrows, zdeg, nsum_out, deg_out,
                     src_v, dst_v, rows_v, ones_v, acc_sh, deg_sh, sem):
    c = lax.axis_index("c")
    s = lax.axis_index("s")
    wid = c * _NS + s
    pltpu.sync_copy(zrows.at[pl.ds(s * _RPT, _RPT)],
                    acc_sh.at[pl.ds(s * _RPT, _RPT)])
    pltpu.sync_copy(zdeg.at[pl.ds(s * _RPT, _RPT)],
                    deg_sh.at[pl.ds(s * _RPT, _RPT)])
    for k in range(_CH // 16):
        ones_v[pl.ds(k * 16, 16)] = jnp.full((16,), 1.0, jnp.float32)
    plsc.subcore_barrier()

    def body(j, carry):
        pltpu.sync_copy(srcs.at[wid, j], src_v)
        pltpu.sync_copy(dsts.at[wid, j], dst_v)
        pltpu.async_copy(table.at[dst_v], rows_v, sem).wait()
        pltpu.sync_copy(rows_v, acc_sh.at[src_v], add=True)
        pltpu.sync_copy(ones_v, deg_sh.at[src_v], add=True)
        return carry

    lax.fori_loop(0, _NCHUNK, body, 0)
    plsc.subcore_barrier()
    pltpu.sync_copy(acc_sh.at[pl.ds(s * _RPT, _RPT)],
                    nsum_out.at[c, pl.ds(s * _RPT, _RPT)])
    pltpu.sync_copy(deg_sh.at[pl.ds(s * _RPT, _RPT)],
                    deg_out.at[c, pl.ds(s * _RPT, _RPT)])


def _sc_seg_body(table, srcs, dsts, zrows, nsum_out,
                 src_v, dst_v, rows_v, acc_sh, sem):
    c = lax.axis_index("c")
    s = lax.axis_index("s")
    wid = c * _NS + s
    pltpu.sync_copy(zrows.at[pl.ds(s * _RPT, _RPT)],
                    acc_sh.at[pl.ds(s * _RPT, _RPT)])
    plsc.subcore_barrier()

    def body(j, carry):
        pltpu.sync_copy(srcs.at[wid, j], src_v)
        pltpu.sync_copy(dsts.at[wid, j], dst_v)
        pltpu.async_copy(table.at[dst_v], rows_v, sem).wait()
        pltpu.sync_copy(rows_v, acc_sh.at[src_v], add=True)
        return carry

    lax.fori_loop(0, _NCHUNK, body, 0)
    plsc.subcore_barrier()
    pltpu.sync_copy(acc_sh.at[pl.ds(s * _RPT, _RPT)],
                    nsum_out.at[c, pl.ds(s * _RPT, _RPT)])


_seg_deg = pl.kernel(
    _sc_seg_deg_body,
    out_type=[jax.ShapeDtypeStruct((_NC, _NP, _D), jnp.float32),
              jax.ShapeDtypeStruct((_NC, _NP), jnp.float32)],
    mesh=_mesh,
    scratch_types=[pltpu.VMEM((_CH,), jnp.int32),
                   pltpu.VMEM((_CH,), jnp.int32),
                   pltpu.VMEM((_CH, _D), jnp.float32),
                   pltpu.VMEM((_CH,), jnp.float32),
                   pltpu.VMEM_SHARED((_NP, _D), jnp.float32),
                   pltpu.VMEM_SHARED((_NP,), jnp.float32),
                   pltpu.SemaphoreType.DMA],
)

_seg = pl.kernel(
    _sc_seg_body,
    out_type=[jax.ShapeDtypeStruct((_NC, _NP, _D), jnp.float32)],
    mesh=_mesh,
    scratch_types=[pltpu.VMEM((_CH,), jnp.int32),
                   pltpu.VMEM((_CH,), jnp.int32),
                   pltpu.VMEM((_CH, _D), jnp.float32),
                   pltpu.VMEM_SHARED((_NP, _D), jnp.float32),
                   pltpu.SemaphoreType.DMA],
)


_DN_NT = (((1,), (1,)), ((), ()))
_DN_NN = (((1,), (0,)), ((), ()))


def _mm3_body(x_ref, wg_ref, wl_ref, ws_ref, hg_ref, hl_ref, hs_ref):
    xb = x_ref[...]
    hg_ref[...] = lax.dot_general(xb, wg_ref[...], _DN_NT,
                                  preferred_element_type=jnp.float32)
    hl_ref[...] = lax.dot_general(xb, wl_ref[...], _DN_NT,
                                  preferred_element_type=jnp.float32)
    hs_ref[...] = lax.dot_general(xb, ws_ref[...], _DN_NT,
                                  preferred_element_type=jnp.float32)


_mm3 = pl.pallas_call(
    _mm3_body,
    grid=(_NBLK,),
    in_specs=[pl.BlockSpec((_RB, _D), lambda i: (i, 0)),
              pl.BlockSpec((_D, _D), lambda i: (0, 0)),
              pl.BlockSpec((_D, _D), lambda i: (0, 0)),
              pl.BlockSpec((_D, _D), lambda i: (0, 0))],
    out_specs=[pl.BlockSpec((_RB, _D), lambda i: (i, 0))] * 3,
    out_shape=[jax.ShapeDtypeStruct((_NP, _D), jnp.float32)] * 3,
)


def _layer_epilogue(hg_ref, hs_ref, nsump_ref, degp_ref, b_ref):
    ns = nsump_ref[...]
    nsum = ns[0] + ns[1]
    dp = degp_ref[...]
    deg = dp[0] + dp[1]
    inv = 1.0 / jnp.maximum(deg, 1.0)
    mask = (deg > 0.0).astype(jnp.float32)
    pre = hg_ref[...] + b_ref[...] + mask * (nsum * inv + hs_ref[...])
    return jnp.where(pre > 0.0, pre, jnp.exp(jnp.minimum(pre, 0.0)) - 1.0)


def _post_mm3_body(hg_ref, hs_ref, nsump_ref, degp_ref, b_ref,
                   wg_ref, wl_ref, ws_ref, hg2_ref, hl2_ref, hs2_ref):
    h1 = _layer_epilogue(hg_ref, hs_ref, nsump_ref, degp_ref, b_ref)
    hg2_ref[...] = lax.dot_general(h1, wg_ref[...], _DN_NT,
                                   preferred_element_type=jnp.float32)
    hl2_ref[...] = lax.dot_general(h1, wl_ref[...], _DN_NT,
                                   preferred_element_type=jnp.float32)
    hs2_ref[...] = lax.dot_general(h1, ws_ref[...], _DN_NT,
                                   preferred_element_type=jnp.float32)


_post_mm3 = pl.pallas_call(
    _post_mm3_body,
    grid=(_NBLK,),
    in_specs=[pl.BlockSpec((_RB, _D), lambda i: (i, 0)),
              pl.BlockSpec((_RB, _D), lambda i: (i, 0)),
              pl.BlockSpec((_NC, _RB, _D), lambda i: (0, i, 0)),
              pl.BlockSpec((_NC, _RB, 1), lambda i: (0, i, 0)),
              pl.BlockSpec((1, _D), lambda i: (0, 0)),
              pl.BlockSpec((_D, _D), lambda i: (0, 0)),
              pl.BlockSpec((_D, _D), lambda i: (0, 0)),
              pl.BlockSpec((_D, _D), lambda i: (0, 0))],
    out_specs=[pl.BlockSpec((_RB, _D), lambda i: (i, 0))] * 3,
    out_shape=[jax.ShapeDtypeStruct((_NP, _D), jnp.float32)] * 3,
)


def _final_body(hg_ref, hs_ref, nsump_ref, degp_ref, b_ref, batch_ref,
                wc_ref, bc_ref, out_ref, sums_s, cnts_s):
    i = pl.program_id(0)

    @pl.when(i == 0)
    def _():
        sums_s[...] = jnp.zeros_like(sums_s)
        cnts_s[...] = jnp.zeros_like(cnts_s)

    h2 = _layer_epilogue(hg_ref, hs_ref, nsump_ref, degp_ref, b_ref)
    bvec = batch_ref[...][0]
    oh = (lax.broadcasted_iota(jnp.int32, (_NG, _RB), 0)
          == jnp.broadcast_to(bvec, (_NG, _RB))).astype(jnp.float32)
    sums_s[...] += lax.dot_general(oh, h2, _DN_NN,
                                   preferred_element_type=jnp.float32)
    cnts_s[...] += jnp.broadcast_to(
        jnp.sum(oh, axis=1, keepdims=True), (_NG, _D))

    @pl.when(i == _NBLK - 1)
    def _():
        g = sums_s[...] / jnp.maximum(cnts_s[...], 1.0)
        out_ref[...] = lax.dot_general(g, wc_ref[...], _DN_NT,
                                       preferred_element_type=jnp.float32) \
            + bc_ref[...]


_final = pl.pallas_call(
    _final_body,
    grid=(_NBLK,),
    in_specs=[pl.BlockSpec((_RB, _D), lambda i: (i, 0)),
              pl.BlockSpec((_RB, _D), lambda i: (i, 0)),
              pl.BlockSpec((_NC, _RB, _D), lambda i: (0, i, 0)),
              pl.BlockSpec((_NC, _RB, 1), lambda i: (0, i, 0)),
              pl.BlockSpec((1, _D), lambda i: (0, 0)),
              pl.BlockSpec((1, 1, _RB), lambda i: (i, 0, 0)),
              pl.BlockSpec((_NCLS, _D), lambda i: (0, 0)),
              pl.BlockSpec((1, _NCLS), lambda i: (0, 0))],
    out_specs=pl.BlockSpec((_NG, _NCLS), lambda i: (0, 0)),
    out_shape=jax.ShapeDtypeStruct((_NG, _NCLS), jnp.float32),
    scratch_shapes=[pltpu.VMEM((_NG, _D), jnp.float32),
                    pltpu.VMEM((_NG, _D), jnp.float32)],
)


def kernel(x, edge_index, batch, Wg1, Wl1, Ws1, b1, Wg2, Wl2, Ws2, b2, Wc, bc):
    src = edge_index[0].astype(jnp.int32)
    dst = edge_index[1].astype(jnp.int32)
    pad = _NW * _EPW - _E
    srcp = jnp.concatenate(
        [src, jnp.full((pad,), _N, jnp.int32)]).reshape(_NW, _NCHUNK, _CH)
    dstp = jnp.concatenate(
        [dst, jnp.zeros((pad,), jnp.int32)]).reshape(_NW, _NCHUNK, _CH)
    xp = jnp.pad(x, ((0, _NP - _N), (0, 0)))
    batchp = jnp.concatenate(
        [batch.astype(jnp.int32),
         jnp.full((_NP - _N,), _NG, jnp.int32)]).reshape(_NBLK, 1, _RB)
    zrows = jnp.zeros((_NP, _D), jnp.float32)
    zdeg = jnp.zeros((_NP,), jnp.float32)
    b1r = b1.reshape(1, _D)
    b2r = b2.reshape(1, _D)
    bcr = bc.reshape(1, _NCLS)

    hg1, hl1, hs1 = _mm3(xp, Wg1, Wl1, Ws1)
    nsum1, degp = _seg_deg(hl1, srcp, dstp, zrows, zdeg)
    degp3 = degp.reshape(_NC, _NP, 1)
    hg2, hl2, hs2 = _post_mm3(hg1, hs1, nsum1, degp3, b1r, Wg2, Wl2, Ws2)
    (nsum2,) = _seg(hl2, srcp, dstp, zrows)
    return _final(hg2, hs2, nsum2, degp3, b2r, batchp, Wc, bcr)


# speedup vs baseline: 1.7140x; 1.0679x over previous
import jax
import jax.numpy as jnp
from jax import lax
from jax.experimental import pallas as pl
from jax.experimental.pallas import tpu as pltpu
from jax.experimental.pallas import tpu_sc as plsc

_N, _E, _D = 10000, 320000, 128
_NG, _NCLS = 64, 10
_NP = 10240
_NC, _NS = 2, 16
_NW = _NC * _NS
_CH = 128
_NCHUNK = 79
_EPW = _NCHUNK * _CH
_RPT = _NP // _NS
_RB = 1024
_NBLK = _NP // _RB

_mesh = plsc.VectorSubcoreMesh(core_axis_name="c", subcore_axis_name="s")


_NIDX = _NCHUNK + 3


def _sc_seg_deg_body(table, srcs, zrows, zdeg, nsum_out, deg_out,
                     src_v, dst_v, rows_v, ones_v, acc_sh, deg_sh, sem):
    c = lax.axis_index("c")
    s = lax.axis_index("s")
    wid = c * _NS + s
    pltpu.sync_copy(zrows.at[pl.ds(s * _RPT, _RPT)],
                    acc_sh.at[pl.ds(s * _RPT, _RPT)])
    pltpu.sync_copy(zdeg.at[pl.ds(s * _RPT, _RPT)],
                    deg_sh.at[pl.ds(s * _RPT, _RPT)])
    for k in range(_CH // 16):
        ones_v[pl.ds(k * 16, 16)] = jnp.full((16,), 1.0, jnp.float32)
    plsc.subcore_barrier()

    def body(j, carry):
        pltpu.sync_copy(srcs.at[wid, j, 0], src_v)
        pltpu.sync_copy(srcs.at[wid, j, 1], dst_v)
        pltpu.async_copy(table.at[dst_v], rows_v, sem).wait()
        pltpu.sync_copy(rows_v, acc_sh.at[src_v], add=True)
        pltpu.sync_copy(ones_v, deg_sh.at[src_v], add=True)
        return carry

    lax.fori_loop(0, _NCHUNK, body, 0)
    plsc.subcore_barrier()
    pltpu.sync_copy(acc_sh.at[pl.ds(s * _RPT, _RPT)],
                    nsum_out.at[c, pl.ds(s * _RPT, _RPT)])
    pltpu.sync_copy(deg_sh.at[pl.ds(s * _RPT, _RPT)],
                    deg_out.at[c, pl.ds(s * _RPT, _RPT)])


def _sc_seg_body(table, srcs, zrows, nsum_out,
                 src_v, dst_v, rows_v, acc_sh, sem):
    c = lax.axis_index("c")
    s = lax.axis_index("s")
    wid = c * _NS + s
    pltpu.sync_copy(zrows.at[pl.ds(s * _RPT, _RPT)],
                    acc_sh.at[pl.ds(s * _RPT, _RPT)])
    plsc.subcore_barrier()

    def body(j, carry):
        pltpu.sync_copy(srcs.at[wid, j, 0], src_v)
        pltpu.sync_copy(srcs.at[wid, j, 1], dst_v)
        pltpu.async_copy(table.at[dst_v], rows_v, sem).wait()
        pltpu.sync_copy(rows_v, acc_sh.at[src_v], add=True)
        return carry

    lax.fori_loop(0, _NCHUNK, body, 0)
    plsc.subcore_barrier()
    pltpu.sync_copy(acc_sh.at[pl.ds(s * _RPT, _RPT)],
                    nsum_out.at[c, pl.ds(s * _RPT, _RPT)])


_seg_deg = pl.kernel(
    _sc_seg_deg_body,
    out_type=[jax.ShapeDtypeStruct((_NC, _NP, _D), jnp.float32),
              jax.ShapeDtypeStruct((_NC, _NP), jnp.float32)],
    mesh=_mesh,
    scratch_types=[pltpu.VMEM((_CH,), jnp.int32),
                   pltpu.VMEM((_CH,), jnp.int32),
                   pltpu.VMEM((_CH, _D), jnp.float32),
                   pltpu.VMEM((_CH,), jnp.float32),
                   pltpu.VMEM_SHARED((_NP, _D), jnp.float32),
                   pltpu.VMEM_SHARED((_NP,), jnp.float32),
                   pltpu.SemaphoreType.DMA],
)

_seg = pl.kernel(
    _sc_seg_body,
    out_type=[jax.ShapeDtypeStruct((_NC, _NP, _D), jnp.float32)],
    mesh=_mesh,
    scratch_types=[pltpu.VMEM((_CH,), jnp.int32),
                   pltpu.VMEM((_CH,), jnp.int32),
                   pltpu.VMEM((_CH, _D), jnp.float32),
                   pltpu.VMEM_SHARED((_NP, _D), jnp.float32),
                   pltpu.SemaphoreType.DMA],
)


_DN_NT = (((1,), (1,)), ((), ()))
_DN_NN = (((1,), (0,)), ((), ()))


def _mm3_body(x_ref, wg_ref, wl_ref, ws_ref, hg_ref, hl_ref, hs_ref):
    xb = x_ref[...]
    hg_ref[...] = lax.dot_general(xb, wg_ref[...], _DN_NT,
                                  preferred_element_type=jnp.float32)
    hl_ref[...] = lax.dot_general(xb, wl_ref[...], _DN_NT,
                                  preferred_element_type=jnp.float32)
    hs_ref[...] = lax.dot_general(xb, ws_ref[...], _DN_NT,
                                  preferred_element_type=jnp.float32)


_mm3 = pl.pallas_call(
    _mm3_body,
    grid=(_NBLK,),
    in_specs=[pl.BlockSpec((_RB, _D), lambda i: (i, 0)),
              pl.BlockSpec((_D, _D), lambda i: (0, 0)),
              pl.BlockSpec((_D, _D), lambda i: (0, 0)),
              pl.BlockSpec((_D, _D), lambda i: (0, 0))],
    out_specs=[pl.BlockSpec((_RB, _D), lambda i: (i, 0))] * 3,
    out_shape=[jax.ShapeDtypeStruct((_NP, _D), jnp.float32)] * 3,
)


def _layer_epilogue(hg_ref, hs_ref, nsump_ref, degp_ref, b_ref):
    ns = nsump_ref[...]
    nsum = ns[0] + ns[1]
    dp = degp_ref[...]
    deg = dp[0] + dp[1]
    inv = 1.0 / jnp.maximum(deg, 1.0)
    mask = (deg > 0.0).astype(jnp.float32)
    pre = hg_ref[...] + b_ref[...] + mask * (nsum * inv + hs_ref[...])
    return jnp.where(pre > 0.0, pre, jnp.exp(jnp.minimum(pre, 0.0)) - 1.0)


def _post_mm3_body(hg_ref, hs_ref, nsump_ref, degp_ref, b_ref,
                   wg_ref, wl_ref, ws_ref, hg2_ref, hl2_ref, hs2_ref):
    h1 = _layer_epilogue(hg_ref, hs_ref, nsump_ref, degp_ref, b_ref)
    hg2_ref[...] = lax.dot_general(h1, wg_ref[...], _DN_NT,
                                   preferred_element_type=jnp.float32)
    hl2_ref[...] = lax.dot_general(h1, wl_ref[...], _DN_NT,
                                   preferred_element_type=jnp.float32)
    hs2_ref[...] = lax.dot_general(h1, ws_ref[...], _DN_NT,
                                   preferred_element_type=jnp.float32)


_post_mm3 = pl.pallas_call(
    _post_mm3_body,
    grid=(_NBLK,),
    in_specs=[pl.BlockSpec((_RB, _D), lambda i: (i, 0)),
              pl.BlockSpec((_RB, _D), lambda i: (i, 0)),
              pl.BlockSpec((_NC, _RB, _D), lambda i: (0, i, 0)),
              pl.BlockSpec((_NC, _RB, 1), lambda i: (0, i, 0)),
              pl.BlockSpec((1, _D), lambda i: (0, 0)),
              pl.BlockSpec((_D, _D), lambda i: (0, 0)),
              pl.BlockSpec((_D, _D), lambda i: (0, 0)),
              pl.BlockSpec((_D, _D), lambda i: (0, 0))],
    out_specs=[pl.BlockSpec((_RB, _D), lambda i: (i, 0))] * 3,
    out_shape=[jax.ShapeDtypeStruct((_NP, _D), jnp.float32)] * 3,
)


def _final_body(hg_ref, hs_ref, nsump_ref, degp_ref, b_ref, batch_ref,
                wc_ref, bc_ref, out_ref, sums_s, cnts_s):
    i = pl.program_id(0)

    @pl.when(i == 0)
    def _():
        sums_s[...] = jnp.zeros_like(sums_s)
        cnts_s[...] = jnp.zeros_like(cnts_s)

    h2 = _layer_epilogue(hg_ref, hs_ref, nsump_ref, degp_ref, b_ref)
    bvec = batch_ref[...][0]
    oh = (lax.broadcasted_iota(jnp.int32, (_NG, _RB), 0)
          == jnp.broadcast_to(bvec, (_NG, _RB))).astype(jnp.float32)
    sums_s[...] += lax.dot_general(oh, h2, _DN_NN,
                                   preferred_element_type=jnp.float32)
    cnts_s[...] += jnp.broadcast_to(
        jnp.sum(oh, axis=1, keepdims=True), (_NG, _D))

    @pl.when(i == _NBLK - 1)
    def _():
        g = sums_s[...] / jnp.maximum(cnts_s[...], 1.0)
        out_ref[...] = lax.dot_general(g, wc_ref[...], _DN_NT,
                                       preferred_element_type=jnp.float32) \
            + bc_ref[...]


_final = pl.pallas_call(
    _final_body,
    grid=(_NBLK,),
    in_specs=[pl.BlockSpec((_RB, _D), lambda i: (i, 0)),
              pl.BlockSpec((_RB, _D), lambda i: (i, 0)),
              pl.BlockSpec((_NC, _RB, _D), lambda i: (0, i, 0)),
              pl.BlockSpec((_NC, _RB, 1), lambda i: (0, i, 0)),
              pl.BlockSpec((1, _D), lambda i: (0, 0)),
              pl.BlockSpec((1, 1, _RB), lambda i: (i, 0, 0)),
              pl.BlockSpec((_NCLS, _D), lambda i: (0, 0)),
              pl.BlockSpec((1, _NCLS), lambda i: (0, 0))],
    out_specs=pl.BlockSpec((_NG, _NCLS), lambda i: (0, 0)),
    out_shape=jax.ShapeDtypeStruct((_NG, _NCLS), jnp.float32),
    scratch_shapes=[pltpu.VMEM((_NG, _D), jnp.float32),
                    pltpu.VMEM((_NG, _D), jnp.float32)],
)


def kernel(x, edge_index, batch, Wg1, Wl1, Ws1, b1, Wg2, Wl2, Ws2, b2, Wc, bc):
    src = edge_index[0].astype(jnp.int32)
    dst = edge_index[1].astype(jnp.int32)
    pad = _NW * _EPW - _E
    srcp = jnp.concatenate(
        [src, jnp.full((pad,), _N, jnp.int32)]).reshape(_NW, _NCHUNK, _CH)
    dstp = jnp.concatenate(
        [dst, jnp.zeros((pad,), jnp.int32)]).reshape(_NW, _NCHUNK, _CH)
    xp = jnp.pad(x, ((0, _NP - _N), (0, 0)))
    batchp = jnp.concatenate(
        [batch.astype(jnp.int32),
         jnp.full((_NP - _N,), _NG, jnp.int32)]).reshape(_NBLK, 1, _RB)
    zrows = jnp.zeros((_NP, _D), jnp.float32)
    zdeg = jnp.zeros((_NP,), jnp.float32)
    b1r = b1.reshape(1, _D)
    b2r = b2.reshape(1, _D)
    bcr = bc.reshape(1, _NCLS)

    hg1, hl1, hs1 = _mm3(xp, Wg1, Wl1, Ws1)
    edges = jnp.stack([srcp, dstp], axis=2)
    nsum1, degp = _seg_deg(hl1, edges, zrows, zdeg)
    degp3 = degp.reshape(_NC, _NP, 1)
    hg2, hl2, hs2 = _post_mm3(hg1, hs1, nsum1, degp3, b1r, Wg2, Wl2, Ws2)
    (nsum2,) = _seg(hl2, edges, zrows)
    return _final(hg2, hs2, nsum2, degp3, b2r, batchp, Wc, bcr)
